# baseline (speedup 1.0000x reference)
import functools

import jax
import jax.numpy as jnp
from jax import lax
from jax.experimental import pallas as pl
from jax.experimental.pallas import tpu as pltpu
from jax.experimental.pallas import tpu_sc as plsc

N = 10000
E = 320000
H = 128
NPAD = 10240
NC = 2
NS = 16
NW = NC * NS
BK = 128
NBLK = 80
CH = 10
NCH = NBLK // CH
E_PAD = NW * NBLK * BK
ROWS_PER_TILE = NPAD // NS
BN = 512

_mesh = plsc.VectorSubcoreMesh(
    core_axis_name="c", subcore_axis_name="s", num_cores=NC, num_subcores=NS
)


@functools.partial(
    pl.kernel,
    out_type=(
        jax.ShapeDtypeStruct((NC, NPAD, H), jnp.float32),
        jax.ShapeDtypeStruct((NW, NPAD), jnp.float32),
    ),
    mesh=_mesh,
    scratch_types=[
        pltpu.VMEM((CH, 2, BK), jnp.int32),
        pltpu.VMEM((2, BK, H), jnp.float32),
        pltpu.VMEM((NPAD,), jnp.float32),
        pltpu.VMEM_SHARED((NPAD, H), jnp.float32),
        pltpu.SemaphoreType.DMA,
        pltpu.SemaphoreType.DMA,
    ],
    compiler_params=pltpu.CompilerParams(needs_layout_passes=False),
)
def _sc_aggregate(h_hbm, idx_hbm, zrows_hbm, zflat_hbm,
                  agg_hbm, deg_hbm,
                  idx_v, gbuf, deg_v, agg_sh, sem0, sem1):
    c = lax.axis_index("c")
    s = lax.axis_index("s")
    wid = s * NC + c

    pltpu.sync_copy(zrows_hbm, agg_sh.at[pl.ds(s * ROWS_PER_TILE, ROWS_PER_TILE)])
    pltpu.sync_copy(zflat_hbm, deg_v)
    plsc.subcore_barrier()

    ones = jnp.ones((16,), jnp.float32)
    sems = (sem0, sem1)

    def chunk_body(k, carry):
        pltpu.sync_copy(idx_hbm.at[wid, pl.ds(k * CH, CH)], idx_v)
        for jj in range(CH):
            pltpu.async_copy(h_hbm.at[idx_v.at[jj, 0]], gbuf.at[0], sems[0]).wait()
            for g in range(BK // 16):
                v = idx_v[jj, 1, pl.ds(g * 16, 16)]
                plsc.addupdate_scatter(deg_v, [v], ones)
            pltpu.sync_copy(gbuf.at[0], agg_sh.at[idx_v.at[jj, 1]], add=True)
        return carry

    lax.fori_loop(0, NCH, chunk_body, 0)
    plsc.subcore_barrier()

    pltpu.sync_copy(
        agg_sh.at[pl.ds(s * ROWS_PER_TILE, ROWS_PER_TILE)],
        agg_hbm.at[c, pl.ds(s * ROWS_PER_TILE, ROWS_PER_TILE)],
    )
    pltpu.sync_copy(deg_v, deg_hbm.at[wid])


def _dot(a, b):
    return jnp.dot(a, b, preferred_element_type=jnp.float32,
                   precision=lax.Precision.HIGHEST)


def _mlp_body(h_ref, a0_ref, a1_ref, deg_ref, w1a_ref, w1b_ref, b1_ref,
              w2_ref, b2_ref, o_ref):
    deg = jnp.sum(deg_ref[...], axis=1, keepdims=True)
    inv = 1.0 / jnp.maximum(deg, 1.0)
    agg = (a0_ref[...] + a1_ref[...]) * inv
    y = _dot(h_ref[...], w1a_ref[...]) + _dot(agg, w1b_ref[...]) + b1_ref[...]
    y = jnp.maximum(y, 0.0)
    z = _dot(y, w2_ref[...]) + b2_ref[...]
    o_ref[...] = jnp.maximum(z, 0.0)


_mlp = pl.pallas_call(
    _mlp_body,
    grid=(NPAD // BN,),
    in_specs=[
        pl.BlockSpec((BN, H), lambda i: (i, 0)),
        pl.BlockSpec((BN, H), lambda i: (i, 0)),
        pl.BlockSpec((BN, H), lambda i: (i, 0)),
        pl.BlockSpec((BN, NW), lambda i: (i, 0)),
        pl.BlockSpec((H, H), lambda i: (0, 0)),
        pl.BlockSpec((H, H), lambda i: (0, 0)),
        pl.BlockSpec((1, H), lambda i: (0, 0)),
        pl.BlockSpec((H, H), lambda i: (0, 0)),
        pl.BlockSpec((1, H), lambda i: (0, 0)),
    ],
    out_specs=pl.BlockSpec((BN, H), lambda i: (i, 0)),
    out_shape=jax.ShapeDtypeStruct((NPAD, H), jnp.float32),
)


def kernel(h, edge_index, W1, b1, W2, b2):
    src = edge_index[0].astype(jnp.int32)
    dst = edge_index[1].astype(jnp.int32)
    pad = E_PAD - E
    src_p = jnp.concatenate([src, jnp.zeros((pad,), jnp.int32)])
    dst_p = jnp.concatenate([dst, jnp.full((pad,), N, jnp.int32)])
    src3 = src_p.reshape(NW, NBLK, BK)
    dst3 = dst_p.reshape(NW, NBLK, BK)
    idx = jnp.stack([src3, dst3], axis=2)
    h_pad = jnp.pad(h, ((0, NPAD - N), (0, 0)))
    zrows = jnp.zeros((ROWS_PER_TILE, H), jnp.float32)
    zflat = jnp.zeros((NPAD,), jnp.float32)

    agg_parts, deg_parts = _sc_aggregate(h_pad, idx, zrows, zflat)

    out = _mlp(h_pad, agg_parts[0], agg_parts[1], deg_parts.T,
               W1[:H], W1[H:], b1.reshape(1, H), W2, b2.reshape(1, H))
    return out[:N]


# speedup vs baseline: 1.1144x; 1.1144x over previous
import functools

import jax
import jax.numpy as jnp
from jax import lax
from jax.experimental import pallas as pl
from jax.experimental.pallas import tpu as pltpu
from jax.experimental.pallas import tpu_sc as plsc

N = 10000
E = 320000
H = 128
NPAD = 10240
NC = 2
NS = 16
NW = NC * NS
BK = 128
NBLK = 80
CH = 10
NCH = NBLK // CH
E_PAD = NW * NBLK * BK
ROWS_PER_TILE = NPAD // NS
BN = 512

_mesh = plsc.VectorSubcoreMesh(
    core_axis_name="c", subcore_axis_name="s", num_cores=NC, num_subcores=NS
)


@functools.partial(
    pl.kernel,
    out_type=(
        jax.ShapeDtypeStruct((NC, NPAD, H), jnp.float32),
        jax.ShapeDtypeStruct((NW, NPAD), jnp.float32),
    ),
    mesh=_mesh,
    scratch_types=[
        pltpu.VMEM((NBLK, BK), jnp.int32),
        pltpu.VMEM((NBLK, BK), jnp.int32),
        pltpu.VMEM((BK, H), jnp.float32),
        pltpu.VMEM((NPAD,), jnp.float32),
        pltpu.VMEM_SHARED((NPAD, H), jnp.float32),
        pltpu.SemaphoreType.DMA,
    ],
    compiler_params=pltpu.CompilerParams(needs_layout_passes=False),
)
def _sc_aggregate(h_hbm, src_hbm, dst_hbm, zrows_hbm, zflat_hbm,
                  agg_hbm, deg_hbm,
                  src_v, dst_v, gbuf, deg_v, agg_sh, sem):
    c = lax.axis_index("c")
    s = lax.axis_index("s")
    wid = s * NC + c

    pltpu.sync_copy(src_hbm.at[wid], src_v)
    pltpu.sync_copy(dst_hbm.at[wid], dst_v)
    pltpu.sync_copy(zrows_hbm, agg_sh.at[pl.ds(s * ROWS_PER_TILE, ROWS_PER_TILE)])
    pltpu.sync_copy(zflat_hbm, deg_v)
    plsc.subcore_barrier()

    ones = jnp.ones((16,), jnp.float32)

    def body(j, carry):
        pltpu.async_copy(h_hbm.at[src_v.at[j]], gbuf, sem).wait()
        pltpu.sync_copy(gbuf, agg_sh.at[dst_v.at[j]], add=True)
        for g in range(BK // 16):
            v = dst_v[j, pl.ds(g * 16, 16)]
            plsc.addupdate_scatter(deg_v, [v], ones)
        return carry

    lax.fori_loop(0, NBLK, body, 0)
    plsc.subcore_barrier()

    pltpu.sync_copy(
        agg_sh.at[pl.ds(s * ROWS_PER_TILE, ROWS_PER_TILE)],
        agg_hbm.at[c, pl.ds(s * ROWS_PER_TILE, ROWS_PER_TILE)],
    )
    pltpu.sync_copy(deg_v, deg_hbm.at[wid])


def _dot(a, b):
    return jnp.dot(a, b, preferred_element_type=jnp.float32,
                   precision=lax.Precision.HIGHEST)


def _mlp_body(h_ref, a0_ref, a1_ref, deg_ref, w1a_ref, w1b_ref, b1_ref,
              w2_ref, b2_ref, o_ref):
    deg = jnp.sum(deg_ref[...], axis=1, keepdims=True)
    inv = 1.0 / jnp.maximum(deg, 1.0)
    agg = (a0_ref[...] + a1_ref[...]) * inv
    y = _dot(h_ref[...], w1a_ref[...]) + _dot(agg, w1b_ref[...]) + b1_ref[...]
    y = jnp.maximum(y, 0.0)
    z = _dot(y, w2_ref[...]) + b2_ref[...]
    o_ref[...] = jnp.maximum(z, 0.0)


_mlp = pl.pallas_call(
    _mlp_body,
    grid=(NPAD // BN,),
    in_specs=[
        pl.BlockSpec((BN, H), lambda i: (i, 0)),
        pl.BlockSpec((BN, H), lambda i: (i, 0)),
        pl.BlockSpec((BN, H), lambda i: (i, 0)),
        pl.BlockSpec((BN, NW), lambda i: (i, 0)),
        pl.BlockSpec((H, H), lambda i: (0, 0)),
        pl.BlockSpec((H, H), lambda i: (0, 0)),
        pl.BlockSpec((1, H), lambda i: (0, 0)),
        pl.BlockSpec((H, H), lambda i: (0, 0)),
        pl.BlockSpec((1, H), lambda i: (0, 0)),
    ],
    out_specs=pl.BlockSpec((BN, H), lambda i: (i, 0)),
    out_shape=jax.ShapeDtypeStruct((NPAD, H), jnp.float32),
)


def kernel(h, edge_index, W1, b1, W2, b2):
    src = edge_index[0].astype(jnp.int32)
    dst = edge_index[1].astype(jnp.int32)
    pad = E_PAD - E
    src_p = jnp.concatenate([src, jnp.zeros((pad,), jnp.int32)])
    dst_p = jnp.concatenate([dst, jnp.full((pad,), N, jnp.int32)])
    src3 = src_p.reshape(NW, NBLK, BK)
    dst3 = dst_p.reshape(NW, NBLK, BK)
    h_pad = jnp.pad(h, ((0, NPAD - N), (0, 0)))
    zrows = jnp.zeros((ROWS_PER_TILE, H), jnp.float32)
    zflat = jnp.zeros((NPAD,), jnp.float32)

    agg_parts, deg_parts = _sc_aggregate(h_pad, src3, dst3, zrows, zflat)

    out = _mlp(h_pad, agg_parts[0], agg_parts[1], deg_parts.T,
               W1[:H], W1[H:], b1.reshape(1, H), W2, b2.reshape(1, H))
    return out[:N]


# speedup vs baseline: 2.3303x; 2.0911x over previous
import functools

import jax
import jax.numpy as jnp
from jax import lax
from jax.experimental import pallas as pl
from jax.experimental.pallas import tpu as pltpu
from jax.experimental.pallas import tpu_sc as plsc

N = 10000
E = 320000
H = 128
NPAD = 10240
NC = 2
NS = 16
NW = NC * NS
BK = 128
NBLK = 80
CH = 10
NCH = NBLK // CH
E_PAD = NW * NBLK * BK
ROWS_PER_TILE = NPAD // NS
BN = 512

_mesh = plsc.VectorSubcoreMesh(
    core_axis_name="c", subcore_axis_name="s", num_cores=NC, num_subcores=NS
)


@functools.partial(
    pl.kernel,
    out_type=(
        jax.ShapeDtypeStruct((NC, NPAD, H), jnp.float32),
        jax.ShapeDtypeStruct((NW, NPAD), jnp.float32),
    ),
    mesh=_mesh,
    scratch_types=[
        pltpu.VMEM((NBLK, BK), jnp.int32),
        pltpu.VMEM((NBLK, BK), jnp.int32),
        pltpu.VMEM((BK, H), jnp.float32),
        pltpu.VMEM((NPAD,), jnp.float32),
        pltpu.VMEM_SHARED((NPAD, H), jnp.float32),
        pltpu.SemaphoreType.DMA,
    ],
    compiler_params=pltpu.CompilerParams(needs_layout_passes=False),
)
def _sc_aggregate(h_hbm, src_hbm, dst_hbm, zrows_hbm, zflat_hbm,
                  agg_hbm, deg_hbm,
                  src_v, dst_v, gbuf, deg_v, agg_sh, sem):
    c = lax.axis_index("c")
    s = lax.axis_index("s")
    wid = s * NC + c

    pltpu.sync_copy(src_hbm.at[wid], src_v)
    pltpu.sync_copy(dst_hbm.at[wid], dst_v)
    pltpu.sync_copy(zrows_hbm, agg_sh.at[pl.ds(s * ROWS_PER_TILE, ROWS_PER_TILE)])
    pltpu.sync_copy(zflat_hbm, deg_v)
    plsc.subcore_barrier()

    ones = jnp.ones((16,), jnp.float32)

    def body(j, carry):
        pltpu.async_copy(h_hbm.at[src_v.at[j]], gbuf, sem).wait()
        pltpu.sync_copy(gbuf, agg_sh.at[dst_v.at[j]], add=True)
        for g in range(BK // 16):
            v = dst_v[j, pl.ds(g * 16, 16)]
            plsc.addupdate_scatter(deg_v, [v], ones)
        return carry

    lax.fori_loop(0, NBLK, body, 0)
    plsc.subcore_barrier()

    pltpu.sync_copy(
        agg_sh.at[pl.ds(s * ROWS_PER_TILE, ROWS_PER_TILE)],
        agg_hbm.at[c, pl.ds(s * ROWS_PER_TILE, ROWS_PER_TILE)],
    )
    pltpu.sync_copy(deg_v, deg_hbm.at[wid])


def _dot(a, b):
    return jnp.dot(a, b, preferred_element_type=jnp.float32,
                   precision=lax.Precision.HIGHEST)


def _mlp_body(h_ref, a0_ref, a1_ref, deg_ref, w1a_ref, w1b_ref, b1_ref,
              w2_ref, b2_ref, o_ref):
    deg = jnp.sum(deg_ref[...], axis=1, keepdims=True)
    inv = 1.0 / jnp.maximum(deg, 1.0)
    agg = (a0_ref[...] + a1_ref[...]) * inv
    y = _dot(h_ref[...], w1a_ref[...]) + _dot(agg, w1b_ref[...]) + b1_ref[...]
    y = jnp.maximum(y, 0.0)
    z = _dot(y, w2_ref[...]) + b2_ref[...]
    o_ref[...] = jnp.maximum(z, 0.0)


_mlp = pl.pallas_call(
    _mlp_body,
    grid=(NPAD // BN,),
    in_specs=[
        pl.BlockSpec((BN, H), lambda i: (i, 0)),
        pl.BlockSpec((BN, H), lambda i: (i, 0)),
        pl.BlockSpec((BN, H), lambda i: (i, 0)),
        pl.BlockSpec((BN, NW), lambda i: (i, 0)),
        pl.BlockSpec((H, H), lambda i: (0, 0)),
        pl.BlockSpec((H, H), lambda i: (0, 0)),
        pl.BlockSpec((1, H), lambda i: (0, 0)),
        pl.BlockSpec((H, H), lambda i: (0, 0)),
        pl.BlockSpec((1, H), lambda i: (0, 0)),
    ],
    out_specs=pl.BlockSpec((BN, H), lambda i: (i, 0)),
    out_shape=jax.ShapeDtypeStruct((NPAD, H), jnp.float32),
)


def kernel(h, edge_index, W1, b1, W2, b2):
    src = edge_index[0].astype(jnp.int32)
    dst = edge_index[1].astype(jnp.int32)
    pad = E_PAD - E
    pad_iota = jnp.arange(pad, dtype=jnp.int32)
    src_p = jnp.concatenate([src, pad_iota % NPAD])
    dst_p = jnp.concatenate([dst, N + (pad_iota % (NPAD - N))])
    src3 = src_p.reshape(NW, NBLK, BK)
    dst3 = dst_p.reshape(NW, NBLK, BK)
    h_pad = jnp.pad(h, ((0, NPAD - N), (0, 0)))
    zrows = jnp.zeros((ROWS_PER_TILE, H), jnp.float32)
    zflat = jnp.zeros((NPAD,), jnp.float32)

    agg_parts, deg_parts = _sc_aggregate(h_pad, src3, dst3, zrows, zflat)

    out = _mlp(h_pad, agg_parts[0], agg_parts[1], deg_parts.T,
               W1[:H], W1[H:], b1.reshape(1, H), W2, b2.reshape(1, H))
    return out[:N]


# speedup vs baseline: 2.3769x; 1.0200x over previous
import functools

import jax
import jax.numpy as jnp
from jax import lax
from jax.experimental import pallas as pl
from jax.experimental.pallas import tpu as pltpu
from jax.experimental.pallas import tpu_sc as plsc

N = 10000
E = 320000
H = 128
NPAD = 10240
NC = 2
NS = 16
NW = NC * NS
BK = 128
NBLK = 80
CH = 10
NCH = NBLK // CH
E_PAD = NW * NBLK * BK
ROWS_PER_TILE = NPAD // NS
BN = 400

_mesh = plsc.VectorSubcoreMesh(
    core_axis_name="c", subcore_axis_name="s", num_cores=NC, num_subcores=NS
)


@functools.partial(
    pl.kernel,
    out_type=(
        jax.ShapeDtypeStruct((NC, NPAD, H), jnp.float32),
        jax.ShapeDtypeStruct((NW, NPAD), jnp.float32),
    ),
    mesh=_mesh,
    scratch_types=[
        pltpu.VMEM((NBLK, BK), jnp.int32),
        pltpu.VMEM((NBLK, BK), jnp.int32),
        pltpu.VMEM((BK, H), jnp.float32),
        pltpu.VMEM((NPAD,), jnp.float32),
        pltpu.VMEM_SHARED((NPAD, H), jnp.float32),
        pltpu.SemaphoreType.DMA,
    ],
    compiler_params=pltpu.CompilerParams(needs_layout_passes=False),
)
def _sc_aggregate(h_hbm, src_hbm, dst_hbm, zrows_hbm, zflat_hbm,
                  agg_hbm, deg_hbm,
                  src_v, dst_v, gbuf, deg_v, agg_sh, sem):
    c = lax.axis_index("c")
    s = lax.axis_index("s")
    wid = s * NC + c

    pltpu.sync_copy(src_hbm.at[wid], src_v)
    pltpu.sync_copy(dst_hbm.at[wid], dst_v)
    pltpu.sync_copy(zrows_hbm, agg_sh.at[pl.ds(s * ROWS_PER_TILE, ROWS_PER_TILE)])
    pltpu.sync_copy(zflat_hbm, deg_v)
    plsc.subcore_barrier()

    ones = jnp.ones((16,), jnp.float32)

    def body(j, carry):
        pltpu.async_copy(h_hbm.at[src_v.at[j]], gbuf, sem).wait()
        pltpu.sync_copy(gbuf, agg_sh.at[dst_v.at[j]], add=True)
        for g in range(BK // 16):
            v = dst_v[j, pl.ds(g * 16, 16)]
            plsc.addupdate_scatter(deg_v, [v], ones)
        return carry

    lax.fori_loop(0, NBLK, body, 0)
    plsc.subcore_barrier()

    pltpu.sync_copy(
        agg_sh.at[pl.ds(s * ROWS_PER_TILE, ROWS_PER_TILE)],
        agg_hbm.at[c, pl.ds(s * ROWS_PER_TILE, ROWS_PER_TILE)],
    )
    pltpu.sync_copy(deg_v, deg_hbm.at[wid])


def _dot(a, b):
    return jnp.dot(a, b, preferred_element_type=jnp.float32,
                   precision=lax.Precision.HIGHEST)


def _mlp_body(h_ref, a0_ref, a1_ref, deg_ref, w1a_ref, w1b_ref, b1_ref,
              w2_ref, b2_ref, o_ref):
    deg = jnp.sum(deg_ref[...], axis=1, keepdims=True)
    inv = 1.0 / jnp.maximum(deg, 1.0)
    agg = (a0_ref[...] + a1_ref[...]) * inv
    y = _dot(h_ref[...], w1a_ref[...]) + _dot(agg, w1b_ref[...]) + b1_ref[...]
    y = jnp.maximum(y, 0.0)
    z = _dot(y, w2_ref[...]) + b2_ref[...]
    o_ref[...] = jnp.maximum(z, 0.0)


_mlp = pl.pallas_call(
    _mlp_body,
    grid=(N // BN,),
    in_specs=[
        pl.BlockSpec((BN, H), lambda i: (i, 0)),
        pl.BlockSpec((BN, H), lambda i: (i, 0)),
        pl.BlockSpec((BN, H), lambda i: (i, 0)),
        pl.BlockSpec((BN, NW), lambda i: (i, 0)),
        pl.BlockSpec((H, H), lambda i: (0, 0)),
        pl.BlockSpec((H, H), lambda i: (0, 0)),
        pl.BlockSpec((1, H), lambda i: (0, 0)),
        pl.BlockSpec((H, H), lambda i: (0, 0)),
        pl.BlockSpec((1, H), lambda i: (0, 0)),
    ],
    out_specs=pl.BlockSpec((BN, H), lambda i: (i, 0)),
    out_shape=jax.ShapeDtypeStruct((N, H), jnp.float32),
)


def kernel(h, edge_index, W1, b1, W2, b2):
    src = edge_index[0].astype(jnp.int32)
    dst = edge_index[1].astype(jnp.int32)
    pad = E_PAD - E
    pad_iota = jnp.arange(pad, dtype=jnp.int32)
    src_p = jnp.concatenate([src, pad_iota % N])
    dst_p = jnp.concatenate([dst, N + (pad_iota % (NPAD - N))])
    src3 = src_p.reshape(NW, NBLK, BK)
    dst3 = dst_p.reshape(NW, NBLK, BK)
    zrows = jnp.zeros((ROWS_PER_TILE, H), jnp.float32)
    zflat = jnp.zeros((NPAD,), jnp.float32)

    agg_parts, deg_parts = _sc_aggregate(h, src3, dst3, zrows, zflat)

    return _mlp(h, agg_parts[0], agg_parts[1], deg_parts.T,
                W1[:H], W1[H:], b1.reshape(1, H), W2, b2.reshape(1, H))


# speedup vs baseline: 3.2237x; 1.3563x over previous
import functools

import jax
import jax.numpy as jnp
from jax import lax
from jax.experimental import pallas as pl
from jax.experimental.pallas import tpu as pltpu
from jax.experimental.pallas import tpu_sc as plsc

N = 10000
E = 320000
H = 128
NPAD = 10240
NC = 2
NS = 16
NW = NC * NS
BK = 128
NBLK = 80
CH = 8
NCH = NBLK // CH
E_PAD = NW * NBLK * BK
ROWS_PER_TILE = NPAD // NS
BN = 400

_mesh = plsc.VectorSubcoreMesh(
    core_axis_name="c", subcore_axis_name="s", num_cores=NC, num_subcores=NS
)


@functools.partial(
    pl.kernel,
    out_type=(
        jax.ShapeDtypeStruct((NC, NPAD, H), jnp.float32),
        jax.ShapeDtypeStruct((NW, NPAD), jnp.float32),
    ),
    mesh=_mesh,
    scratch_types=[
        pltpu.VMEM((CH, 2, BK), jnp.int32),
        pltpu.VMEM((CH, 2, BK), jnp.int32),
        pltpu.VMEM((BK, H), jnp.float32),
        pltpu.VMEM((BK, H), jnp.float32),
        pltpu.VMEM((NPAD,), jnp.float32),
        pltpu.VMEM_SHARED((NPAD, H), jnp.float32),
        pltpu.SemaphoreType.DMA,
        pltpu.SemaphoreType.DMA,
        pltpu.SemaphoreType.DMA,
        pltpu.SemaphoreType.DMA,
    ],
    compiler_params=pltpu.CompilerParams(needs_layout_passes=False),
)
def _sc_aggregate(h_hbm, idx_hbm, zrows_hbm, zflat_hbm,
                  agg_hbm, deg_hbm,
                  ibuf0, ibuf1, gbuf0, gbuf1, deg_v, agg_sh,
                  isem0, isem1, sem0, sem1):
    c = lax.axis_index("c")
    s = lax.axis_index("s")
    wid = s * NC + c

    pltpu.sync_copy(zrows_hbm, agg_sh.at[pl.ds(s * ROWS_PER_TILE, ROWS_PER_TILE)])
    pltpu.sync_copy(zflat_hbm, deg_v)
    plsc.subcore_barrier()

    ones = jnp.ones((16,), jnp.float32)
    ibufs = (ibuf0, ibuf1)
    isems = (isem0, isem1)
    gbufs = (gbuf0, gbuf1)
    gsems = (sem0, sem1)

    def _stage(k, p):
        return pltpu.async_copy(
            idx_hbm.at[wid, pl.ds(k * CH, CH)], ibufs[p], isems[p])

    def _gather(ib, jj, b):
        return pltpu.async_copy(h_hbm.at[ib.at[jj, 0]], gbufs[b], gsems[b])

    _stage(0, 0).wait()
    _gather(ibuf0, 0, 0)

    def body(k2, carry):
        for p in range(2):
            k = 2 * k2 + p
            ib = ibufs[p]
            for jj in range(CH):
                b = jj % 2
                if jj == 1:
                    @pl.when(k + 1 < NCH)
                    def _():
                        _stage(k + 1, 1 - p)
                if jj < CH - 1:
                    _gather(ib, jj + 1, 1 - b)
                else:
                    @pl.when(k + 1 < NCH)
                    def _():
                        pltpu.make_async_copy(
                            idx_hbm.at[wid, pl.ds((k + 1) * CH, CH)],
                            ibufs[1 - p], isems[1 - p]).wait()
                        _gather(ibufs[1 - p], 0, 1 - b)
                pltpu.make_async_copy(
                    h_hbm.at[ib.at[jj, 0]], gbufs[b], gsems[b]).wait()
                pltpu.sync_copy(gbufs[b], agg_sh.at[ib.at[jj, 1]], add=True)
                for g in range(BK // 16):
                    v = ib[jj, 1, pl.ds(g * 16, 16)]
                    plsc.addupdate_scatter(deg_v, [v], ones)
        return carry

    lax.fori_loop(0, NCH // 2, body, 0)
    plsc.subcore_barrier()

    pltpu.sync_copy(
        agg_sh.at[pl.ds(s * ROWS_PER_TILE, ROWS_PER_TILE)],
        agg_hbm.at[c, pl.ds(s * ROWS_PER_TILE, ROWS_PER_TILE)],
    )
    pltpu.sync_copy(deg_v, deg_hbm.at[wid])


def _dot(a, b):
    return jnp.dot(a, b, preferred_element_type=jnp.float32,
                   precision=lax.Precision.HIGHEST)


def _mlp_body(h_ref, a0_ref, a1_ref, deg_ref, w1a_ref, w1b_ref, b1_ref,
              w2_ref, b2_ref, o_ref):
    deg = jnp.sum(deg_ref[...], axis=1, keepdims=True)
    inv = 1.0 / jnp.maximum(deg, 1.0)
    agg = (a0_ref[...] + a1_ref[...]) * inv
    y = _dot(h_ref[...], w1a_ref[...]) + _dot(agg, w1b_ref[...]) + b1_ref[...]
    y = jnp.maximum(y, 0.0)
    z = _dot(y, w2_ref[...]) + b2_ref[...]
    o_ref[...] = jnp.maximum(z, 0.0)


_mlp = pl.pallas_call(
    _mlp_body,
    grid=(N // BN,),
    in_specs=[
        pl.BlockSpec((BN, H), lambda i: (i, 0)),
        pl.BlockSpec((BN, H), lambda i: (i, 0)),
        pl.BlockSpec((BN, H), lambda i: (i, 0)),
        pl.BlockSpec((BN, NW), lambda i: (i, 0)),
        pl.BlockSpec((H, H), lambda i: (0, 0)),
        pl.BlockSpec((H, H), lambda i: (0, 0)),
        pl.BlockSpec((1, H), lambda i: (0, 0)),
        pl.BlockSpec((H, H), lambda i: (0, 0)),
        pl.BlockSpec((1, H), lambda i: (0, 0)),
    ],
    out_specs=pl.BlockSpec((BN, H), lambda i: (i, 0)),
    out_shape=jax.ShapeDtypeStruct((N, H), jnp.float32),
)


def kernel(h, edge_index, W1, b1, W2, b2):
    src = edge_index[0].astype(jnp.int32)
    dst = edge_index[1].astype(jnp.int32)
    pad = E_PAD - E
    pad_iota = jnp.arange(pad, dtype=jnp.int32)
    src_p = jnp.concatenate([src, pad_iota % N])
    dst_p = jnp.concatenate([dst, N + (pad_iota % (NPAD - N))])
    src3 = src_p.reshape(NW, NBLK, BK)
    dst3 = dst_p.reshape(NW, NBLK, BK)
    idx = jnp.stack([src3, dst3], axis=2)
    zrows = jnp.zeros((ROWS_PER_TILE, H), jnp.float32)
    zflat = jnp.zeros((NPAD,), jnp.float32)

    agg_parts, deg_parts = _sc_aggregate(h, idx, zrows, zflat)

    return _mlp(h, agg_parts[0], agg_parts[1], deg_parts.T,
                W1[:H], W1[H:], b1.reshape(1, H), W2, b2.reshape(1, H))


# speedup vs baseline: 3.4324x; 1.0648x over previous
import functools

import jax
import jax.numpy as jnp
from jax import lax
from jax.experimental import pallas as pl
from jax.experimental.pallas import tpu as pltpu
from jax.experimental.pallas import tpu_sc as plsc

N = 10000
E = 320000
H = 128
NPAD = 10240
NC = 2
NS = 16
NW = NC * NS
BK = 125
NBLK = 80
CH = 8
NCH = NBLK // CH
ROWS_PER_TILE = NPAD // NS
BN = 400

_mesh = plsc.VectorSubcoreMesh(
    core_axis_name="c", subcore_axis_name="s", num_cores=NC, num_subcores=NS
)


@functools.partial(
    pl.kernel,
    out_type=(
        jax.ShapeDtypeStruct((NC, NPAD, H), jnp.float32),
        jax.ShapeDtypeStruct((NW, NPAD), jnp.float32),
    ),
    mesh=_mesh,
    scratch_types=[
        pltpu.VMEM((CH, BK), jnp.int32),
        pltpu.VMEM((CH, BK), jnp.int32),
        pltpu.VMEM((CH, BK), jnp.int32),
        pltpu.VMEM((CH, BK), jnp.int32),
        pltpu.VMEM((BK, H), jnp.float32),
        pltpu.VMEM((BK, H), jnp.float32),
        pltpu.VMEM((NPAD,), jnp.float32),
        pltpu.VMEM_SHARED((NPAD, H), jnp.float32),
        pltpu.SemaphoreType.DMA,
        pltpu.SemaphoreType.DMA,
        pltpu.SemaphoreType.DMA,
        pltpu.SemaphoreType.DMA,
    ],
    compiler_params=pltpu.CompilerParams(needs_layout_passes=False),
)
def _sc_aggregate(h_hbm, idx_hbm, zrows_hbm, zflat_hbm,
                  agg_hbm, deg_hbm,
                  isbuf0, isbuf1, idbuf0, idbuf1, gbuf0, gbuf1, deg_v, agg_sh,
                  isem0, isem1, sem0, sem1):
    c = lax.axis_index("c")
    s = lax.axis_index("s")
    wid = s * NC + c

    pltpu.sync_copy(zrows_hbm, agg_sh.at[pl.ds(s * ROWS_PER_TILE, ROWS_PER_TILE)])
    pltpu.sync_copy(zflat_hbm, deg_v)
    plsc.subcore_barrier()

    ones = jnp.ones((16,), jnp.float32)
    lane = lax.iota(jnp.int32, 16)
    tail_mask = lane >= 3
    isbufs = (isbuf0, isbuf1)
    idbufs = (idbuf0, idbuf1)
    isems = (isem0, isem1)
    gbufs = (gbuf0, gbuf1)
    gsems = (sem0, sem1)

    def _stage(k, p):
        pltpu.async_copy(idx_hbm.at[0, wid, pl.ds(k * CH, CH)], isbufs[p], isems[p])
        pltpu.async_copy(idx_hbm.at[1, wid, pl.ds(k * CH, CH)], idbufs[p], isems[p])

    def _stage_wait(k, p):
        pltpu.make_async_copy(
            idx_hbm.at[0, wid, pl.ds(k * CH, CH)], isbufs[p], isems[p]).wait()
        pltpu.make_async_copy(
            idx_hbm.at[1, wid, pl.ds(k * CH, CH)], idbufs[p], isems[p]).wait()

    def _gather(ib, jj, b):
        return pltpu.async_copy(h_hbm.at[ib.at[jj]], gbufs[b], gsems[b])

    _stage(0, 0)
    _stage_wait(0, 0)
    _gather(isbuf0, 0, 0)

    def body(k2, carry):
        for p in range(2):
            k = 2 * k2 + p
            isb = isbufs[p]
            idb = idbufs[p]
            for jj in range(CH):
                b = jj % 2
                if jj == 1:
                    @pl.when(k + 1 < NCH)
                    def _():
                        _stage(k + 1, 1 - p)
                if jj < CH - 1:
                    _gather(isb, jj + 1, 1 - b)
                else:
                    @pl.when(k + 1 < NCH)
                    def _():
                        _stage_wait(k + 1, 1 - p)
                        _gather(isbufs[1 - p], 0, 1 - b)
                pltpu.make_async_copy(
                    h_hbm.at[isb.at[jj]], gbufs[b], gsems[b]).wait()
                pltpu.sync_copy(gbufs[b], agg_sh.at[idb.at[jj]], add=True)
                for g in range(7):
                    v = idb[jj, pl.ds(g * 16, 16)]
                    plsc.addupdate_scatter(deg_v, [v], ones)
                v = idb[jj, pl.ds(BK - 16, 16)]
                plsc.addupdate_scatter(deg_v, [v], ones, mask=tail_mask)
        return carry

    lax.fori_loop(0, NCH // 2, body, 0)
    plsc.subcore_barrier()

    pltpu.sync_copy(
        agg_sh.at[pl.ds(s * ROWS_PER_TILE, ROWS_PER_TILE)],
        agg_hbm.at[c, pl.ds(s * ROWS_PER_TILE, ROWS_PER_TILE)],
    )
    pltpu.sync_copy(deg_v, deg_hbm.at[wid])


def _dot(a, b):
    return jnp.dot(a, b, preferred_element_type=jnp.float32,
                   precision=lax.Precision.HIGHEST)


def _mlp_body(h_ref, a0_ref, a1_ref, deg_ref, w1a_ref, w1b_ref, b1_ref,
              w2_ref, b2_ref, o_ref):
    deg = jnp.sum(deg_ref[...], axis=1, keepdims=True)
    inv = 1.0 / jnp.maximum(deg, 1.0)
    agg = (a0_ref[...] + a1_ref[...]) * inv
    y = _dot(h_ref[...], w1a_ref[...]) + _dot(agg, w1b_ref[...]) + b1_ref[...]
    y = jnp.maximum(y, 0.0)
    z = _dot(y, w2_ref[...]) + b2_ref[...]
    o_ref[...] = jnp.maximum(z, 0.0)


_mlp = pl.pallas_call(
    _mlp_body,
    grid=(N // BN,),
    in_specs=[
        pl.BlockSpec((BN, H), lambda i: (i, 0)),
        pl.BlockSpec((BN, H), lambda i: (i, 0)),
        pl.BlockSpec((BN, H), lambda i: (i, 0)),
        pl.BlockSpec((BN, NW), lambda i: (i, 0)),
        pl.BlockSpec((H, H), lambda i: (0, 0)),
        pl.BlockSpec((H, H), lambda i: (0, 0)),
        pl.BlockSpec((1, H), lambda i: (0, 0)),
        pl.BlockSpec((H, H), lambda i: (0, 0)),
        pl.BlockSpec((1, H), lambda i: (0, 0)),
    ],
    out_specs=pl.BlockSpec((BN, H), lambda i: (i, 0)),
    out_shape=jax.ShapeDtypeStruct((N, H), jnp.float32),
)


def kernel(h, edge_index, W1, b1, W2, b2):
    idx = edge_index.astype(jnp.int32).reshape(2, NW, NBLK, BK)
    zrows = jnp.zeros((ROWS_PER_TILE, H), jnp.float32)
    zflat = jnp.zeros((NPAD,), jnp.float32)

    agg_parts, deg_parts = _sc_aggregate(h, idx, zrows, zflat)

    return _mlp(h, agg_parts[0], agg_parts[1], deg_parts.T,
                W1[:H], W1[H:], b1.reshape(1, H), W2, b2.reshape(1, H))


# speedup vs baseline: 3.5440x; 1.0325x over previous
import functools

import jax
import jax.numpy as jnp
from jax import lax
from jax.experimental import pallas as pl
from jax.experimental.pallas import tpu as pltpu
from jax.experimental.pallas import tpu_sc as plsc

N = 10000
E = 320000
H = 128
NPAD = 10240
NC = 2
NS = 16
NW = NC * NS
BK = 125
NBLK = 80
CH = 8
NCH = NBLK // CH
ROWS_PER_TILE = NPAD // NS
BN = 400

_mesh = plsc.VectorSubcoreMesh(
    core_axis_name="c", subcore_axis_name="s", num_cores=NC, num_subcores=NS
)


@functools.partial(
    pl.kernel,
    out_type=(
        jax.ShapeDtypeStruct((NC, NPAD, H), jnp.float32),
        jax.ShapeDtypeStruct((NW, NPAD), jnp.float32),
    ),
    mesh=_mesh,
    scratch_types=[
        pltpu.VMEM((CH, BK), jnp.int32),
        pltpu.VMEM((CH, BK), jnp.int32),
        pltpu.VMEM((CH, BK), jnp.int32),
        pltpu.VMEM((CH, BK), jnp.int32),
        pltpu.VMEM((BK, H), jnp.float32),
        pltpu.VMEM((BK, H), jnp.float32),
        pltpu.VMEM((NPAD,), jnp.float32),
        pltpu.VMEM_SHARED((NPAD, H), jnp.float32),
        pltpu.SemaphoreType.DMA,
        pltpu.SemaphoreType.DMA,
        pltpu.SemaphoreType.DMA,
        pltpu.SemaphoreType.DMA,
    ],
    compiler_params=pltpu.CompilerParams(needs_layout_passes=False),
)
def _sc_aggregate(h_hbm, idx_hbm, zrows_hbm, zflat_hbm,
                  agg_hbm, deg_hbm,
                  isbuf0, isbuf1, idbuf0, idbuf1, gbuf0, gbuf1, deg_v, agg_sh,
                  isem0, isem1, sem0, sem1):
    c = lax.axis_index("c")
    s = lax.axis_index("s")
    wid = s * NC + c

    pltpu.sync_copy(zrows_hbm, agg_sh.at[pl.ds(s * ROWS_PER_TILE, ROWS_PER_TILE)])
    pltpu.sync_copy(zflat_hbm, deg_v)
    plsc.subcore_barrier()

    ones = jnp.ones((16,), jnp.float32)
    lane = lax.iota(jnp.int32, 16)
    tail_mask = lane >= 3
    isbufs = (isbuf0, isbuf1)
    idbufs = (idbuf0, idbuf1)
    isems = (isem0, isem1)
    gbufs = (gbuf0, gbuf1)
    gsems = (sem0, sem1)

    def _stage(k, p):
        pltpu.async_copy(idx_hbm.at[0, wid, pl.ds(k * CH, CH)], isbufs[p], isems[p])
        pltpu.async_copy(idx_hbm.at[1, wid, pl.ds(k * CH, CH)], idbufs[p], isems[p])

    def _stage_wait(k, p):
        pltpu.make_async_copy(
            idx_hbm.at[0, wid, pl.ds(k * CH, CH)], isbufs[p], isems[p]).wait()
        pltpu.make_async_copy(
            idx_hbm.at[1, wid, pl.ds(k * CH, CH)], idbufs[p], isems[p]).wait()

    def _gather(ib, jj, b):
        return pltpu.async_copy(h_hbm.at[ib.at[jj]], gbufs[b], gsems[b])

    _stage(0, 0)
    _stage_wait(0, 0)
    _gather(isbuf0, 0, 0)

    def body(k2, carry):
        for p in range(2):
            k = 2 * k2 + p
            isb = isbufs[p]
            idb = idbufs[p]
            for jj in range(CH):
                b = jj % 2
                if jj == 1:
                    @pl.when(k + 1 < NCH)
                    def _():
                        _stage(k + 1, 1 - p)
                if jj < CH - 1:
                    _gather(isb, jj + 1, 1 - b)
                else:
                    @pl.when(k + 1 < NCH)
                    def _():
                        _stage_wait(k + 1, 1 - p)
                        _gather(isbufs[1 - p], 0, 1 - b)
                pltpu.make_async_copy(
                    h_hbm.at[isb.at[jj]], gbufs[b], gsems[b]).wait()
                pltpu.sync_copy(gbufs[b], agg_sh.at[idb.at[jj]], add=True)
                for g in range(7):
                    v = idb[jj, pl.ds(g * 16, 16)]
                    plsc.addupdate_scatter(deg_v, [v], ones)
                v = idb[jj, pl.ds(BK - 16, 16)]
                plsc.addupdate_scatter(deg_v, [v], ones, mask=tail_mask)
        return carry

    lax.fori_loop(0, NCH // 2, body, 0)
    plsc.subcore_barrier()

    pltpu.sync_copy(
        agg_sh.at[pl.ds(s * ROWS_PER_TILE, ROWS_PER_TILE)],
        agg_hbm.at[c, pl.ds(s * ROWS_PER_TILE, ROWS_PER_TILE)],
    )
    pltpu.sync_copy(deg_v, deg_hbm.at[wid])


def _dot(a, b):
    return jnp.dot(a, b, preferred_element_type=jnp.float32,
                   precision=lax.Precision.HIGHEST)


def _mlp_body(h_ref, a0_ref, a1_ref, deg_ref, w1a_ref, w1b_ref, b1_ref,
              w2_ref, b2_ref, o_ref):
    deg = jnp.sum(deg_ref[...], axis=1, keepdims=True)
    inv = 1.0 / jnp.maximum(deg, 1.0)
    agg = (a0_ref[0] + a1_ref[0]) * inv
    y = _dot(h_ref[...], w1a_ref[...]) + _dot(agg, w1b_ref[...]) + b1_ref[...]
    y = jnp.maximum(y, 0.0)
    z = _dot(y, w2_ref[...]) + b2_ref[...]
    o_ref[...] = jnp.maximum(z, 0.0)


_mlp = pl.pallas_call(
    _mlp_body,
    grid=(N // BN,),
    in_specs=[
        pl.BlockSpec((BN, H), lambda i: (i, 0)),
        pl.BlockSpec((1, BN, H), lambda i: (0, i, 0)),
        pl.BlockSpec((1, BN, H), lambda i: (1, i, 0)),
        pl.BlockSpec((BN, NW), lambda i: (i, 0)),
        pl.BlockSpec((H, H), lambda i: (0, 0)),
        pl.BlockSpec((H, H), lambda i: (0, 0)),
        pl.BlockSpec((1, H), lambda i: (0, 0)),
        pl.BlockSpec((H, H), lambda i: (0, 0)),
        pl.BlockSpec((1, H), lambda i: (0, 0)),
    ],
    out_specs=pl.BlockSpec((BN, H), lambda i: (i, 0)),
    out_shape=jax.ShapeDtypeStruct((N, H), jnp.float32),
)


def kernel(h, edge_index, W1, b1, W2, b2):
    idx = edge_index.astype(jnp.int32).reshape(2, NW, NBLK, BK)
    zrows = jnp.zeros((ROWS_PER_TILE, H), jnp.float32)
    zflat = jnp.zeros((NPAD,), jnp.float32)

    agg_parts, deg_parts = _sc_aggregate(h, idx, zrows, zflat)

    return _mlp(h, agg_parts, agg_parts, deg_parts.T,
                W1[:H], W1[H:], b1.reshape(1, H), W2, b2.reshape(1, H))


# speedup vs baseline: 3.7506x; 1.0583x over previous
import functools

import jax
import jax.numpy as jnp
from jax import lax
from jax.experimental import pallas as pl
from jax.experimental.pallas import tpu as pltpu
from jax.experimental.pallas import tpu_sc as plsc

N = 10000
E = 320000
H = 128
NPAD = 10240
NC = 2
NS = 16
NW = NC * NS
BK = 125
NBLK = 80
CH = 8
NCH = NBLK // CH
ROWS_PER_TILE = NPAD // NS
BN = 400

_mesh = plsc.VectorSubcoreMesh(
    core_axis_name="c", subcore_axis_name="s", num_cores=NC, num_subcores=NS
)


@functools.partial(
    pl.kernel,
    out_type=(
        jax.ShapeDtypeStruct((NC, NPAD, H), jnp.float32),
        jax.ShapeDtypeStruct((NW, NPAD), jnp.float32),
    ),
    mesh=_mesh,
    scratch_types=[
        pltpu.VMEM((CH, BK), jnp.int32),
        pltpu.VMEM((CH, BK), jnp.int32),
        pltpu.VMEM((CH, BK), jnp.int32),
        pltpu.VMEM((CH, BK), jnp.int32),
        pltpu.VMEM((BK, H), jnp.float32),
        pltpu.VMEM((BK, H), jnp.float32),
        pltpu.VMEM((NPAD,), jnp.float32),
        pltpu.VMEM_SHARED((NPAD, H), jnp.float32),
        pltpu.SemaphoreType.DMA,
        pltpu.SemaphoreType.DMA,
        pltpu.SemaphoreType.DMA,
        pltpu.SemaphoreType.DMA,
    ],
    compiler_params=pltpu.CompilerParams(needs_layout_passes=False),
)
def _sc_aggregate(h_hbm, idx_hbm, zrows_hbm, zflat_hbm,
                  agg_hbm, deg_hbm,
                  isbuf0, isbuf1, idbuf0, idbuf1, gbuf0, gbuf1, deg_v, agg_sh,
                  isem0, isem1, sem0, sem1):
    c = lax.axis_index("c")
    s = lax.axis_index("s")
    wid = s * NC + c

    pltpu.sync_copy(zrows_hbm, agg_sh.at[pl.ds(s * ROWS_PER_TILE, ROWS_PER_TILE)])
    pltpu.sync_copy(zflat_hbm, deg_v)
    plsc.subcore_barrier()

    ones = jnp.ones((16,), jnp.float32)
    lane = lax.iota(jnp.int32, 16)
    tail_mask = lane >= 3
    isbufs = (isbuf0, isbuf1)
    idbufs = (idbuf0, idbuf1)
    isems = (isem0, isem1)
    gbufs = (gbuf0, gbuf1)
    gsems = (sem0, sem1)

    def _stage(k, p):
        pltpu.async_copy(idx_hbm.at[0, wid, pl.ds(k * CH, CH)], isbufs[p], isems[p])
        pltpu.async_copy(idx_hbm.at[1, wid, pl.ds(k * CH, CH)], idbufs[p], isems[p])

    def _stage_wait(k, p):
        pltpu.make_async_copy(
            idx_hbm.at[0, wid, pl.ds(k * CH, CH)], isbufs[p], isems[p]).wait()
        pltpu.make_async_copy(
            idx_hbm.at[1, wid, pl.ds(k * CH, CH)], idbufs[p], isems[p]).wait()

    def _gather(ib, jj, b):
        return pltpu.async_copy(h_hbm.at[ib.at[jj]], gbufs[b], gsems[b])

    _stage(0, 0)
    _stage_wait(0, 0)
    _gather(isbuf0, 0, 0)

    def body(k2, carry):
        for p in range(2):
            k = 2 * k2 + p
            isb = isbufs[p]
            idb = idbufs[p]
            for jj in range(CH):
                b = jj % 2
                if jj == 1:
                    @pl.when(k + 1 < NCH)
                    def _():
                        _stage(k + 1, 1 - p)
                if jj < CH - 1:
                    _gather(isb, jj + 1, 1 - b)
                else:
                    @pl.when(k + 1 < NCH)
                    def _():
                        _stage_wait(k + 1, 1 - p)
                        _gather(isbufs[1 - p], 0, 1 - b)
                pltpu.make_async_copy(
                    h_hbm.at[isb.at[jj]], gbufs[b], gsems[b]).wait()
                pltpu.sync_copy(gbufs[b], agg_sh.at[idb.at[jj]], add=True)
                for g in range(7):
                    v = idb[jj, pl.ds(g * 16, 16)]
                    plsc.addupdate_scatter(deg_v, [v], ones)
                v = idb[jj, pl.ds(BK - 16, 16)]
                plsc.addupdate_scatter(deg_v, [v], ones, mask=tail_mask)
        return carry

    lax.fori_loop(0, NCH // 2, body, 0)
    plsc.subcore_barrier()

    pltpu.sync_copy(
        agg_sh.at[pl.ds(s * ROWS_PER_TILE, ROWS_PER_TILE)],
        agg_hbm.at[c, pl.ds(s * ROWS_PER_TILE, ROWS_PER_TILE)],
    )
    pltpu.sync_copy(deg_v, deg_hbm.at[wid])


def _dot(a, b):
    return jnp.dot(a, b, preferred_element_type=jnp.float32,
                   precision=lax.Precision.DEFAULT)


def _mlp_body(h_ref, a0_ref, a1_ref, deg_ref, w1a_ref, w1b_ref, b1_ref,
              w2_ref, b2_ref, o_ref):
    deg = jnp.sum(deg_ref[...], axis=1, keepdims=True)
    inv = 1.0 / jnp.maximum(deg, 1.0)
    agg = (a0_ref[0] + a1_ref[0]) * inv
    y = _dot(h_ref[...], w1a_ref[...]) + _dot(agg, w1b_ref[...]) + b1_ref[...]
    y = jnp.maximum(y, 0.0)
    z = _dot(y, w2_ref[...]) + b2_ref[...]
    o_ref[...] = jnp.maximum(z, 0.0)


_mlp = pl.pallas_call(
    _mlp_body,
    grid=(N // BN,),
    in_specs=[
        pl.BlockSpec((BN, H), lambda i: (i, 0)),
        pl.BlockSpec((1, BN, H), lambda i: (0, i, 0)),
        pl.BlockSpec((1, BN, H), lambda i: (1, i, 0)),
        pl.BlockSpec((BN, NW), lambda i: (i, 0)),
        pl.BlockSpec((H, H), lambda i: (0, 0)),
        pl.BlockSpec((H, H), lambda i: (0, 0)),
        pl.BlockSpec((1, H), lambda i: (0, 0)),
        pl.BlockSpec((H, H), lambda i: (0, 0)),
        pl.BlockSpec((1, H), lambda i: (0, 0)),
    ],
    out_specs=pl.BlockSpec((BN, H), lambda i: (i, 0)),
    out_shape=jax.ShapeDtypeStruct((N, H), jnp.float32),
)


def kernel(h, edge_index, W1, b1, W2, b2):
    idx = edge_index.astype(jnp.int32).reshape(2, NW, NBLK, BK)
    zrows = jnp.zeros((ROWS_PER_TILE, H), jnp.float32)
    zflat = jnp.zeros((NPAD,), jnp.float32)

    agg_parts, deg_parts = _sc_aggregate(h, idx, zrows, zflat)

    return _mlp(h, agg_parts, agg_parts, deg_parts.T,
                W1[:H], W1[H:], b1.reshape(1, H), W2, b2.reshape(1, H))
